# baseline (speedup 1.0000x reference)
import jax
import jax.numpy as jnp
from jax import lax
from jax.experimental import pallas as pl
from jax.experimental.pallas import tpu as pltpu
from jax.experimental.pallas import tpu_sc as plsc

VOCAB = 100000
EMBED = 64
BATCH = 1024
CTX = 20

NC = 2
NS = 16
NW = NC * NS
ROWS_PER_W = BATCH * CTX // NW
CHUNK = 128
NCHUNK = ROWS_PER_W // CHUNK


def _sc_gather_body(x_hbm, table_hbm, e_hbm, idx_v, rows_v, sem):
    wid = lax.axis_index("s") * NC + lax.axis_index("c")
    base = wid * ROWS_PER_W
    pltpu.sync_copy(x_hbm.at[wid], idx_v)
    copies = [
        pltpu.async_copy(
            table_hbm.at[idx_v.at[j]],
            rows_v.at[pl.ds(j * CHUNK, CHUNK)],
            sem,
        )
        for j in range(NCHUNK)
    ]
    for c in copies:
        c.wait()
    pltpu.sync_copy(rows_v, e_hbm.at[pl.ds(base, ROWS_PER_W)])


import functools


@functools.cache
def _make_sc_gather():
    return pl.kernel(
        _sc_gather_body,
        out_type=jax.ShapeDtypeStruct((BATCH * CTX, EMBED), jnp.float32),
        mesh=plsc.VectorSubcoreMesh(core_axis_name="c", subcore_axis_name="s"),
        compiler_params=pltpu.CompilerParams(use_tc_tiling_on_sc=False),
        scratch_types=[
            pltpu.VMEM((NCHUNK, CHUNK), jnp.int32),
            pltpu.VMEM((ROWS_PER_W, EMBED), jnp.float32),
            pltpu.SemaphoreType.DMA,
        ],
    )


def _pool_body(e_ref, h_ref):
    acc = jnp.zeros((BATCH, EMBED), jnp.float32)
    for j in range(CTX):
        row = e_ref[:, j, :]
        sumsq = jnp.sum(row * row, axis=-1, keepdims=True)
        norm = jnp.sqrt(sumsq)
        scale = jnp.where(norm > 1.0, 1.0 / (norm + 1e-7), 1.0)
        acc = acc + row * scale
    h_ref[...] = acc * (1.0 / CTX)


def _pool(e3):
    return pl.pallas_call(
        _pool_body,
        out_shape=jax.ShapeDtypeStruct((BATCH, EMBED), jnp.float32),
    )(e3)


VT = 1024
GRID_V = (VOCAB + VT - 1) // VT


def _matmul_body(h_ref, w_ref, b_ref, out_ref):
    out = lax.dot_general(
        h_ref[...], w_ref[...],
        (((1,), (1,)), ((), ())),
        preferred_element_type=jnp.float32,
    )
    out_ref[...] = out + b_ref[...]


def _matmul(h, w, b2):
    return pl.pallas_call(
        _matmul_body,
        out_shape=jax.ShapeDtypeStruct((BATCH, VOCAB), jnp.float32),
        grid=(GRID_V,),
        in_specs=[
            pl.BlockSpec((BATCH, EMBED), lambda i: (0, 0)),
            pl.BlockSpec((VT, EMBED), lambda i: (i, 0)),
            pl.BlockSpec((1, VT), lambda i: (0, i)),
        ],
        out_specs=pl.BlockSpec((BATCH, VT), lambda i: (0, i)),
    )(h, w, b2)


def kernel(x, emb_table, W, b):
    x_flat = x.astype(jnp.int32).reshape(NW, NCHUNK, CHUNK)
    e = _make_sc_gather()(x_flat, emb_table)
    h = _pool(e.reshape(BATCH, CTX, EMBED))
    return _matmul(h, W, b.reshape(1, VOCAB))


# speedup vs baseline: 1.0714x; 1.0714x over previous
import jax
import jax.numpy as jnp
from jax import lax
from jax.experimental import pallas as pl
from jax.experimental.pallas import tpu as pltpu
from jax.experimental.pallas import tpu_sc as plsc

VOCAB = 100000
EMBED = 64
BATCH = 1024
CTX = 20

NC = 2
NS = 16
NW = NC * NS
ROWS_PER_W = BATCH * CTX // NW
CHUNK = 128
NCHUNK = ROWS_PER_W // CHUNK


def _sc_gather_body(x_hbm, table_hbm, e_hbm, idx_v, rows_v, sem):
    wid = lax.axis_index("s") * NC + lax.axis_index("c")
    base = wid * ROWS_PER_W
    pltpu.sync_copy(x_hbm.at[wid], idx_v)
    copies = [
        pltpu.async_copy(
            table_hbm.at[idx_v.at[j]],
            rows_v.at[pl.ds(j * CHUNK, CHUNK)],
            sem,
        )
        for j in range(NCHUNK)
    ]
    for c in copies:
        c.wait()
    pltpu.sync_copy(rows_v, e_hbm.at[pl.ds(base, ROWS_PER_W)])


import functools


@functools.cache
def _make_sc_gather():
    return pl.kernel(
        _sc_gather_body,
        out_type=jax.ShapeDtypeStruct((BATCH * CTX, EMBED), jnp.float32),
        mesh=plsc.VectorSubcoreMesh(core_axis_name="c", subcore_axis_name="s"),
        compiler_params=pltpu.CompilerParams(use_tc_tiling_on_sc=False),
        scratch_types=[
            pltpu.VMEM((NCHUNK, CHUNK), jnp.int32),
            pltpu.VMEM((ROWS_PER_W, EMBED), jnp.float32),
            pltpu.SemaphoreType.DMA,
        ],
    )


def _pool_body(e_ref, h_ref):
    acc = jnp.zeros((BATCH, EMBED), jnp.float32)
    for j in range(CTX):
        row = e_ref[:, j, :]
        sumsq = jnp.sum(row * row, axis=-1, keepdims=True)
        norm = jnp.sqrt(sumsq)
        scale = jnp.where(norm > 1.0, 1.0 / (norm + 1e-7), 1.0)
        acc = acc + row * scale
    h_ref[...] = acc * (1.0 / CTX)


def _pool(e3):
    return pl.pallas_call(
        _pool_body,
        out_shape=jax.ShapeDtypeStruct((BATCH, EMBED), jnp.float32),
    )(e3)


VT = 1024
GRID_V = (VOCAB + VT - 1) // VT


def _matmul_body(h_ref, wt_ref, b_ref, out_ref):
    out = lax.dot_general(
        h_ref[...], wt_ref[...],
        (((1,), (0,)), ((), ())),
        preferred_element_type=jnp.float32,
    )
    out_ref[...] = out + b_ref[...]


def _matmul(h, wt, b2):
    return pl.pallas_call(
        _matmul_body,
        out_shape=jax.ShapeDtypeStruct((BATCH, VOCAB), jnp.float32),
        grid=(GRID_V,),
        in_specs=[
            pl.BlockSpec((BATCH, EMBED), lambda i: (0, 0)),
            pl.BlockSpec((EMBED, VT), lambda i: (0, i)),
            pl.BlockSpec((1, VT), lambda i: (0, i)),
        ],
        out_specs=pl.BlockSpec((BATCH, VT), lambda i: (0, i)),
    )(h, wt, b2)


def kernel(x, emb_table, W, b):
    x_flat = x.astype(jnp.int32).reshape(NW, NCHUNK, CHUNK)
    e = _make_sc_gather()(x_flat, emb_table)
    h = _pool(e.reshape(BATCH, CTX, EMBED))
    return _matmul(h, W.T, b.reshape(1, VOCAB))
